# baseline (speedup 1.0000x reference)
import functools

import jax
import jax.numpy as jnp
from jax import lax
from jax.experimental import pallas as pl
from jax.experimental.pallas import tpu as pltpu
from jax.experimental.pallas import tpu_sc as plsc

N = 10000
E = 320000
D = 128
H = 16
K = 3

NC = 2
NS = 16
L = 16
NW = NC * NS
EPW = E // NW
GRP = EPW // L
ROWS_PT = N // NS
CB = 2000
NBLK = EPW // CB
ROWS_IO = 1000
NS_IO = N // ROWS_IO

_mesh = plsc.VectorSubcoreMesh(core_axis_name="c", subcore_axis_name="s")

_f32 = jnp.float32
_i32 = jnp.int32


def _worker(c, s):
    return s * NC + c


def _zero16():
    return jnp.zeros((L,), _f32)


_GDN = lax.GatherDimensionNumbers(offset_dims=(), collapsed_slice_dims=(0,),
                                  start_index_map=(0,))


def _splat(v, j):
    return lax.broadcast_in_dim(lax.squeeze(lax.slice(v, (j,), (j + 1,)), (0,)),
                                (L,), ())


EPC = E // NS
DCH = 2000
NDT = N // DCH


def _sc_gcnnorm_body(row_hbm, col_hbm, w_hbm, out_hbm,
                     colv2, wv2, rowv, colv, wv, disv, nrmv, zinit,
                     degp, disp, acc, acc2):
    c = lax.axis_index("c")
    s = lax.axis_index("s")
    dbase = s * EPC
    pltpu.sync_copy(col_hbm.at[pl.ds(dbase, EPC)], colv2)
    pltpu.sync_copy(w_hbm.at[pl.ds(dbase, EPC)], wv2)

    @pl.when(s == 0)
    def _():
        def zb(i, carry):
            zinit[pl.ds(i * L, L)] = _zero16()
            return carry
        lax.fori_loop(0, N // L, zb, 0)
        pltpu.sync_copy(zinit, acc)

    plsc.subcore_barrier()
    pltpu.sync_copy(wv2, acc.at[colv2], add=True)
    plsc.subcore_barrier()

    @pl.when(s < NDT)
    def _():
        pltpu.sync_copy(acc.at[pl.ds(s * DCH, DCH)], degp)

        def newton(i, carry):
            x = degp[pl.ds(i * L, L)]
            pos = x > 0.0
            iv = lax.bitcast_convert_type(x, _i32)
            y = lax.bitcast_convert_type(
                jnp.int32(0x5F3759DF) - lax.shift_right_logical(iv, 1), _f32)
            for _ in range(3):
                y = y * (1.5 - 0.5 * x * y * y)
            disp[pl.ds(i * L, L)] = jnp.where(pos, y, 0.0)
            return carry

        lax.fori_loop(0, DCH // L, newton, 0)
        pltpu.sync_copy(disp, acc2.at[pl.ds(s * DCH, DCH)])

    plsc.subcore_barrier()
    pltpu.sync_copy(acc2, disv)

    base = _worker(c, s) * EPW
    pltpu.sync_copy(row_hbm.at[pl.ds(base, EPW)], rowv)
    pltpu.sync_copy(col_hbm.at[pl.ds(base, EPW)], colv)
    pltpu.sync_copy(w_hbm.at[pl.ds(base, EPW)], wv)

    def body(g, carry):
        r16 = rowv[pl.ds(g * L, L)]
        c16 = colv[pl.ds(g * L, L)]
        a16 = wv[pl.ds(g * L, L)]
        dr = plsc.load_gather(disv, [r16])
        dc = plsc.load_gather(disv, [c16])
        nrmv[pl.ds(g * L, L)] = a16 * dr * dc
        return carry

    lax.fori_loop(0, GRP, body, 0)
    pltpu.sync_copy(nrmv, out_hbm.at[pl.ds(base, EPW)])


_sc_gcnnorm = pl.kernel(
    _sc_gcnnorm_body,
    out_type=jax.ShapeDtypeStruct((E,), _f32),
    mesh=_mesh,
    compiler_params=pltpu.CompilerParams(needs_layout_passes=False),
    scratch_types=[
        pltpu.VMEM((EPC,), _i32),
        pltpu.VMEM((EPC,), _f32),
        pltpu.VMEM((EPW,), _i32),
        pltpu.VMEM((EPW,), _i32),
        pltpu.VMEM((EPW,), _f32),
        pltpu.VMEM((N,), _f32),
        pltpu.VMEM((EPW,), _f32),
        pltpu.VMEM((N,), _f32),
        pltpu.VMEM((DCH,), _f32),
        pltpu.VMEM((DCH,), _f32),
        pltpu.VMEM_SHARED((N,), _f32),
        pltpu.VMEM_SHARED((N,), _f32),
    ],
)


NBUF = 2


def _sc_prop16_body(z_hbm, row_hbm, col_hbm, nrm_hbm, out_hbm,
                    rowv, colv, nrmv, gbuf, zrow, gsem, ssem0, ssem1, acc):
    c = lax.axis_index("c")
    s = lax.axis_index("s")
    base = _worker(c, s) * EPW
    for b in range(NBLK):
        pltpu.sync_copy(row_hbm.at[pl.ds(base + b * CB, CB)], rowv.at[b])
        pltpu.sync_copy(col_hbm.at[pl.ds(base + b * CB, CB)], colv.at[b])
    pltpu.sync_copy(nrm_hbm.at[pl.ds(base, EPW)], nrmv)

    @pl.when(s < NS_IO)
    def _():
        def zb(i, carry):
            zrow[i, :] = _zero16()
            return carry
        lax.fori_loop(0, ROWS_IO, zb, 0)
        pltpu.sync_copy(zrow, acc.at[pl.ds(s * ROWS_IO, ROWS_IO)])

    plsc.subcore_barrier()

    ssems = (ssem0, ssem1)
    sd = [None] * NBLK
    gd = [None] * NBLK
    gd[0] = pltpu.async_copy(z_hbm.at[rowv.at[0]], gbuf.at[0], gsem)
    for b in range(NBLK):
        nb = b % NBUF
        gd[b].wait()
        if b + 1 < NBLK:
            if b + 1 >= NBUF:
                sd[b + 1 - NBUF].wait()
            gd[b + 1] = pltpu.async_copy(z_hbm.at[rowv.at[b + 1]],
                                         gbuf.at[(b + 1) % NBUF], gsem)

        @plsc.parallel_loop(0, CB // L)
        def scale(g):
            n16 = nrmv[pl.ds(b * CB + g * L, L)]
            for j in range(L):
                e = g * L + j
                gbuf[nb, e, :] = gbuf[nb, e, :] * _splat(n16, j)

        sd[b] = pltpu.async_copy(gbuf.at[nb], acc.at[colv.at[b]],
                                 ssems[nb], add=True)
    for b in range(NBLK - NBUF, NBLK):
        sd[b].wait()
    plsc.subcore_barrier()

    @pl.when(s < NS_IO)
    def _():
        pltpu.sync_copy(acc.at[pl.ds(s * ROWS_IO, ROWS_IO)],
                        out_hbm.at[c, pl.ds(s * ROWS_IO, ROWS_IO)])


_sc_prop16 = pl.kernel(
    _sc_prop16_body,
    out_type=jax.ShapeDtypeStruct((NC, N, H), _f32),
    mesh=_mesh,
    compiler_params=pltpu.CompilerParams(needs_layout_passes=False,
                                         use_tc_tiling_on_sc=False),
    scratch_types=[
        pltpu.VMEM((NBLK, CB), _i32),
        pltpu.VMEM((NBLK, CB), _i32),
        pltpu.VMEM((EPW,), _f32),
        pltpu.VMEM((NBUF, CB, H), _f32),
        pltpu.VMEM((ROWS_IO, H), _f32),
        pltpu.SemaphoreType.DMA,
        pltpu.SemaphoreType.DMA,
        pltpu.SemaphoreType.DMA,
        pltpu.VMEM_SHARED((N, H), _f32),
    ],
)


EPT3 = E // NS
G3 = EPT3 // L


def _sc_layer3_body(s3_hbm, s2_hbm, s1_hbm, s0b_hbm, row_hbm, col_hbm,
                    nrm_hbm, out_hbm, zv, qv, rowv, colv, nrmv, msgv,
                    zinit, acc):
    c = lax.axis_index("c")
    s = lax.axis_index("s")

    @pl.when(c == 0)
    def _():
        base = s * EPT3
        pltpu.sync_copy(row_hbm.at[pl.ds(base, EPT3)], rowv)
        pltpu.sync_copy(col_hbm.at[pl.ds(base, EPT3)], colv)
        pltpu.sync_copy(nrm_hbm.at[pl.ds(base, EPT3)], nrmv)
        pltpu.sync_copy(s3_hbm, zv)

        @pl.when(s == 0)
        def _():
            def zb(i, carry):
                zinit[pl.ds(i * L, L)] = _zero16()
                return carry
            lax.fori_loop(0, N // L, zb, 0)
            pltpu.sync_copy(zinit, acc)

        plsc.subcore_barrier()

        for hop, nxt_hbm in enumerate((s2_hbm, s1_hbm, s0b_hbm)):
            @plsc.parallel_loop(0, G3)
            def mk(g):
                r16 = rowv[pl.ds(g * L, L)]
                n16 = nrmv[pl.ds(g * L, L)]
                msgv[pl.ds(g * L, L)] = plsc.load_gather(zv, [r16]) * n16

            pltpu.sync_copy(msgv, acc.at[colv], add=True)
            plsc.subcore_barrier()
            if hop < 2:
                pltpu.sync_copy(acc, zv)
                pltpu.sync_copy(nxt_hbm, qv)

                def addq(g, carry):
                    zv[pl.ds(g * L, L)] = (zv[pl.ds(g * L, L)]
                                           + qv[pl.ds(g * L, L)])
                    return carry
                lax.fori_loop(0, N // L, addq, 0)
                plsc.subcore_barrier()

                @pl.when(s == 0)
                def _():
                    pltpu.sync_copy(zinit, acc)

                plsc.subcore_barrier()
            else:
                @pl.when(s == 0)
                def _():
                    pltpu.sync_copy(acc, zv)
                    pltpu.sync_copy(nxt_hbm, qv)

                    def addq(g, carry):
                        zv[pl.ds(g * L, L)] = (zv[pl.ds(g * L, L)]
                                               + qv[pl.ds(g * L, L)])
                        return carry
                    lax.fori_loop(0, N // L, addq, 0)
                    pltpu.sync_copy(zv, out_hbm)


_sc_layer3 = pl.kernel(
    _sc_layer3_body,
    out_type=jax.ShapeDtypeStruct((N,), _f32),
    mesh=_mesh,
    compiler_params=pltpu.CompilerParams(needs_layout_passes=False),
    scratch_types=[
        pltpu.VMEM((N,), _f32),
        pltpu.VMEM((N,), _f32),
        pltpu.VMEM((EPT3,), _i32),
        pltpu.VMEM((EPT3,), _i32),
        pltpu.VMEM((EPT3,), _f32),
        pltpu.VMEM((EPT3,), _f32),
        pltpu.VMEM((N,), _f32),
        pltpu.VMEM_SHARED((N,), _f32),
    ],
)


def _tc_proj1_body(x_ref, w_ref, u0, u1, u2, u3):
    x = x_ref[...]
    for k, o in enumerate((u0, u1, u2, u3)):
        o[...] = jnp.dot(x, w_ref[k], preferred_element_type=_f32)


def _tc_proj1(x, W1):
    return pl.pallas_call(
        _tc_proj1_body,
        out_shape=tuple(jax.ShapeDtypeStruct((N, H), _f32) for _ in range(K + 1)),
    )(x, W1)


def _tc_comb_body(u_ref, p_ref, o_ref):
    o_ref[...] = u_ref[...] + p_ref[0] + p_ref[1]


def _tc_comb(u, p):
    return pl.pallas_call(
        _tc_comb_body,
        out_shape=jax.ShapeDtypeStruct((N, H), _f32),
    )(u, p)


def _tc_act_proj_body(u0_ref, p_ref, b_ref, a_ref, w_ref, o0, o1, o2, o3):
    h = u0_ref[...] + p_ref[0] + p_ref[1] + b_ref[...]
    a = a_ref[0, 0]
    h = jnp.where(h >= 0.0, h, a * h)
    for k, o in enumerate((o0, o1, o2, o3)):
        o[...] = jnp.dot(h, w_ref[k], preferred_element_type=_f32)


def _tc_act_proj(u0, p, b, a, W):
    return pl.pallas_call(
        _tc_act_proj_body,
        out_shape=tuple(jax.ShapeDtypeStruct((N, H), _f32) for _ in range(K + 1)),
    )(u0, p, b, a, W)


def _tc_act_proj3_body(v0_ref, p_ref, b_ref, a_ref, w_ref, b3_ref,
                       o0, o1, o2, o3):
    h = v0_ref[...] + p_ref[0] + p_ref[1] + b_ref[...]
    a = a_ref[0, 0]
    h = jnp.where(h >= 0.0, h, a * h)
    for k, o in enumerate((o0, o1, o2, o3)):
        o[...] = jnp.dot(h, w_ref[k, :, 0], preferred_element_type=_f32)
    o0[...] = o0[...] + b3_ref[0, 0]


def _tc_act_proj3(v0, p, b, a, W, b3):
    return pl.pallas_call(
        _tc_act_proj3_body,
        out_shape=tuple(jax.ShapeDtypeStruct((N,), _f32) for _ in range(K + 1)),
    )(v0, p, b, a, W, b3)


def kernel(x, edge_index, edge_attr, W1, b1, W2, b2, W3, b3, a1, a2):
    row = edge_index[0]
    col = edge_index[1]
    a1r = a1.reshape(1, 1)
    a2r = a2.reshape(1, 1)
    b3r = b3.reshape(1, 1)

    nrm = _sc_gcnnorm(row, col, edge_attr)
    u0, u1, u2, u3 = _tc_proj1(x, W1)

    p = _sc_prop16(u3, row, col, nrm)
    z = _tc_comb(u2, p)
    p = _sc_prop16(z, row, col, nrm)
    z = _tc_comb(u1, p)
    p = _sc_prop16(z, row, col, nrm)
    v0, v1, v2, v3 = _tc_act_proj(u0, p, b1, a1r, W2)

    p = _sc_prop16(v3, row, col, nrm)
    z = _tc_comb(v2, p)
    p = _sc_prop16(z, row, col, nrm)
    z = _tc_comb(v1, p)
    p = _sc_prop16(z, row, col, nrm)
    s0b, s1, s2, s3 = _tc_act_proj3(v0, p, b2, a2r, W3, b3r)

    out = _sc_layer3(s3, s2, s1, s0b, row, col, nrm)
    return out.reshape(N, 1)


# speedup vs baseline: 1.2188x; 1.2188x over previous
import functools

import jax
import jax.numpy as jnp
from jax import lax
from jax.experimental import pallas as pl
from jax.experimental.pallas import tpu as pltpu
from jax.experimental.pallas import tpu_sc as plsc

N = 10000
E = 320000
D = 128
H = 16
K = 3

NC = 2
NS = 16
L = 16
NW = NC * NS
EPW = E // NW
GRP = EPW // L
ROWS_PT = N // NS
CB = 2000
NBLK = EPW // CB
ROWS_IO = 1000
NS_IO = N // ROWS_IO

_mesh = plsc.VectorSubcoreMesh(core_axis_name="c", subcore_axis_name="s")

_f32 = jnp.float32
_i32 = jnp.int32


def _worker(c, s):
    return s * NC + c


def _zero16():
    return jnp.zeros((L,), _f32)


_GDN = lax.GatherDimensionNumbers(offset_dims=(), collapsed_slice_dims=(0,),
                                  start_index_map=(0,))


def _splat(v, j):
    return lax.broadcast_in_dim(lax.squeeze(lax.slice(v, (j,), (j + 1,)), (0,)),
                                (L,), ())


EPC = E // NS
DCH = 2000
NDT = N // DCH


def _sc_gcnnorm_body(row_hbm, col_hbm, w_hbm, out_hbm,
                     colv2, wv2, rowv, colv, wv, disv, nrmv, zinit,
                     degp, disp, acc, acc2):
    c = lax.axis_index("c")
    s = lax.axis_index("s")
    dbase = s * EPC
    pltpu.sync_copy(col_hbm.at[pl.ds(dbase, EPC)], colv2)
    pltpu.sync_copy(w_hbm.at[pl.ds(dbase, EPC)], wv2)

    @pl.when(s == 0)
    def _():
        def zb(i, carry):
            zinit[pl.ds(i * L, L)] = _zero16()
            return carry
        lax.fori_loop(0, N // L, zb, 0)
        pltpu.sync_copy(zinit, acc)

    plsc.subcore_barrier()
    pltpu.sync_copy(wv2, acc.at[colv2], add=True)
    plsc.subcore_barrier()

    @pl.when(s < NDT)
    def _():
        pltpu.sync_copy(acc.at[pl.ds(s * DCH, DCH)], degp)

        def newton(i, carry):
            x = degp[pl.ds(i * L, L)]
            pos = x > 0.0
            iv = lax.bitcast_convert_type(x, _i32)
            y = lax.bitcast_convert_type(
                jnp.int32(0x5F3759DF) - lax.shift_right_logical(iv, 1), _f32)
            for _ in range(3):
                y = y * (1.5 - 0.5 * x * y * y)
            disp[pl.ds(i * L, L)] = jnp.where(pos, y, 0.0)
            return carry

        lax.fori_loop(0, DCH // L, newton, 0)
        pltpu.sync_copy(disp, acc2.at[pl.ds(s * DCH, DCH)])

    plsc.subcore_barrier()
    pltpu.sync_copy(acc2, disv)

    base = _worker(c, s) * EPW
    pltpu.sync_copy(row_hbm.at[pl.ds(base, EPW)], rowv)
    pltpu.sync_copy(col_hbm.at[pl.ds(base, EPW)], colv)
    pltpu.sync_copy(w_hbm.at[pl.ds(base, EPW)], wv)

    def body(g, carry):
        r16 = rowv[pl.ds(g * L, L)]
        c16 = colv[pl.ds(g * L, L)]
        a16 = wv[pl.ds(g * L, L)]
        dr = plsc.load_gather(disv, [r16])
        dc = plsc.load_gather(disv, [c16])
        nrmv[pl.ds(g * L, L)] = a16 * dr * dc
        return carry

    lax.fori_loop(0, GRP, body, 0)
    pltpu.sync_copy(nrmv, out_hbm.at[pl.ds(base, EPW)])


_sc_gcnnorm = pl.kernel(
    _sc_gcnnorm_body,
    out_type=jax.ShapeDtypeStruct((E,), _f32),
    mesh=_mesh,
    compiler_params=pltpu.CompilerParams(needs_layout_passes=False),
    scratch_types=[
        pltpu.VMEM((EPC,), _i32),
        pltpu.VMEM((EPC,), _f32),
        pltpu.VMEM((EPW,), _i32),
        pltpu.VMEM((EPW,), _i32),
        pltpu.VMEM((EPW,), _f32),
        pltpu.VMEM((N,), _f32),
        pltpu.VMEM((EPW,), _f32),
        pltpu.VMEM((N,), _f32),
        pltpu.VMEM((DCH,), _f32),
        pltpu.VMEM((DCH,), _f32),
        pltpu.VMEM_SHARED((N,), _f32),
        pltpu.VMEM_SHARED((N,), _f32),
    ],
)


NBUF = 2
HROWS = ROWS_IO // 2


def _magic(slot):
    return (jnp.arange(L, dtype=_i32) * jnp.int32(1103515245)
            + jnp.int32(slot * 1000003 + 777777))


def _sig(flags_ref, slot, flagv):
    flagv[...] = _magic(slot)
    pltpu.sync_copy(flagv, flags_ref.at[slot])


def _poll(flags_ref, slot, flagv):
    magic = _magic(slot)

    def cond(carry):
        return carry == 0

    def body(carry):
        pltpu.sync_copy(flags_ref.at[slot], flagv)
        ok = jnp.all(flagv[...] == magic)
        return jnp.where(ok, 1, 0).astype(_i32)

    lax.while_loop(cond, body, jnp.int32(0))


def _sc_layer16_body(z0_hbm, u2_hbm, u1_hbm, row_hbm, col_hbm, nrm_hbm,
                     t_out, zbuf, p1buf, flags,
                     rowv, colv, nrmv, gbuf, zrow, flagv,
                     gsem, ssem0, ssem1, acc):
    c = lax.axis_index("c")
    s = lax.axis_index("s")
    base = _worker(c, s) * EPW
    for b in range(NBLK):
        pltpu.sync_copy(row_hbm.at[pl.ds(base + b * CB, CB)], rowv.at[b])
        pltpu.sync_copy(col_hbm.at[pl.ds(base + b * CB, CB)], colv.at[b])
    pltpu.sync_copy(nrm_hbm.at[pl.ds(base, EPW)], nrmv)

    def zrow_init():
        def zb(i, carry):
            zrow[i, :] = _zero16()
            return carry
        lax.fori_loop(0, ROWS_IO, zb, 0)

    o = s * ROWS_IO

    @pl.when(s < NS_IO)
    def _():
        zrow_init()
        pltpu.sync_copy(zrow, acc.at[pl.ds(o, ROWS_IO)])

    plsc.subcore_barrier()

    for hop in range(3):
        zsrc = z0_hbm if hop == 0 else zbuf
        ssems = (ssem0, ssem1)
        sd = [None] * NBLK
        gd = [None] * NBLK
        gd[0] = pltpu.async_copy(zsrc.at[rowv.at[0]], gbuf.at[0], gsem)
        for b in range(NBLK):
            nb = b % NBUF
            gd[b].wait()
            if b + 1 < NBLK:
                if b + 1 >= NBUF:
                    sd[b + 1 - NBUF].wait()
                gd[b + 1] = pltpu.async_copy(zsrc.at[rowv.at[b + 1]],
                                             gbuf.at[(b + 1) % NBUF], gsem)

            @plsc.parallel_loop(0, CB // L)
            def scale(g):
                n16 = nrmv[pl.ds(b * CB + g * L, L)]
                for j in range(L):
                    e = g * L + j
                    gbuf[nb, e, :] = gbuf[nb, e, :] * _splat(n16, j)

            sd[b] = pltpu.async_copy(gbuf.at[nb], acc.at[colv.at[b]],
                                     ssems[nb], add=True)
        for b in range(NBLK - NBUF, NBLK):
            sd[b].wait()
        plsc.subcore_barrier()

        @pl.when(c == 1)
        def _():
            @pl.when(s < NS_IO)
            def _():
                pltpu.sync_copy(acc.at[pl.ds(o, ROWS_IO)],
                                p1buf.at[pl.ds(o, ROWS_IO)])
                if hop < 2:
                    pltpu.sync_copy(zrow, acc.at[pl.ds(o, ROWS_IO)])
            plsc.subcore_barrier()

            @pl.when(s == 0)
            def _():
                _sig(flags, 2 * hop, flagv)
                if hop < 2:
                    _poll(flags, 2 * hop + 1, flagv)
            plsc.subcore_barrier()

        @pl.when(c == 0)
        def _():
            @pl.when(s == 0)
            def _():
                _poll(flags, 2 * hop, flagv)
            plsc.subcore_barrier()

            @pl.when(s < NS_IO)
            def _():
                sl = pl.ds(o, ROWS_IO)
                pltpu.sync_copy(acc.at[sl], gbuf.at[0, pl.ds(0, ROWS_IO)])
                pltpu.sync_copy(p1buf.at[sl], gbuf.at[0, pl.ds(ROWS_IO, ROWS_IO)])
                if hop < 2:
                    ucomb = u2_hbm if hop == 0 else u1_hbm
                    pltpu.sync_copy(ucomb.at[sl], gbuf.at[1, pl.ds(0, ROWS_IO)])

                    def comb(r, carry):
                        gbuf[1, ROWS_IO + r, :] = (gbuf[0, r, :]
                                                   + gbuf[0, ROWS_IO + r, :]
                                                   + gbuf[1, r, :])
                        return carry
                    lax.fori_loop(0, ROWS_IO, comb, 0)
                    pltpu.sync_copy(gbuf.at[1, pl.ds(ROWS_IO, ROWS_IO)],
                                    zbuf.at[sl])
                    pltpu.sync_copy(zrow, acc.at[sl])
                else:
                    def comb2(r, carry):
                        gbuf[1, ROWS_IO + r, :] = (gbuf[0, r, :]
                                                   + gbuf[0, ROWS_IO + r, :])
                        return carry
                    lax.fori_loop(0, ROWS_IO, comb2, 0)
                    pltpu.sync_copy(gbuf.at[1, pl.ds(ROWS_IO, ROWS_IO)],
                                    t_out.at[sl])
            plsc.subcore_barrier()

            if hop < 2:
                @pl.when(s == 0)
                def _():
                    _sig(flags, 2 * hop + 1, flagv)


_sc_layer16 = pl.kernel(
    _sc_layer16_body,
    out_type=(
        jax.ShapeDtypeStruct((N, H), _f32),
        jax.ShapeDtypeStruct((N, H), _f32),
        jax.ShapeDtypeStruct((N, H), _f32),
        jax.ShapeDtypeStruct((6, L), _i32),
    ),
    mesh=_mesh,
    compiler_params=pltpu.CompilerParams(needs_layout_passes=False,
                                         use_tc_tiling_on_sc=False),
    scratch_types=[
        pltpu.VMEM((NBLK, CB), _i32),
        pltpu.VMEM((NBLK, CB), _i32),
        pltpu.VMEM((EPW,), _f32),
        pltpu.VMEM((NBUF, CB, H), _f32),
        pltpu.VMEM((ROWS_IO, H), _f32),
        pltpu.VMEM((L,), _i32),
        pltpu.SemaphoreType.DMA,
        pltpu.SemaphoreType.DMA,
        pltpu.SemaphoreType.DMA,
        pltpu.VMEM_SHARED((N, H), _f32),
    ],
)


EPT3 = E // NS
G3 = EPT3 // L


def _sc_layer3_body(s3_hbm, s2_hbm, s1_hbm, s0b_hbm, row_hbm, col_hbm,
                    nrm_hbm, out_hbm, zv, qv, rowv, colv, nrmv, msgv,
                    zinit, acc):
    c = lax.axis_index("c")
    s = lax.axis_index("s")

    @pl.when(c == 0)
    def _():
        base = s * EPT3
        pltpu.sync_copy(row_hbm.at[pl.ds(base, EPT3)], rowv)
        pltpu.sync_copy(col_hbm.at[pl.ds(base, EPT3)], colv)
        pltpu.sync_copy(nrm_hbm.at[pl.ds(base, EPT3)], nrmv)
        pltpu.sync_copy(s3_hbm, zv)

        @pl.when(s == 0)
        def _():
            def zb(i, carry):
                zinit[pl.ds(i * L, L)] = _zero16()
                return carry
            lax.fori_loop(0, N // L, zb, 0)
            pltpu.sync_copy(zinit, acc)

        plsc.subcore_barrier()

        for hop, nxt_hbm in enumerate((s2_hbm, s1_hbm, s0b_hbm)):
            @plsc.parallel_loop(0, G3)
            def mk(g):
                r16 = rowv[pl.ds(g * L, L)]
                n16 = nrmv[pl.ds(g * L, L)]
                msgv[pl.ds(g * L, L)] = plsc.load_gather(zv, [r16]) * n16

            pltpu.sync_copy(msgv, acc.at[colv], add=True)
            plsc.subcore_barrier()
            if hop < 2:
                pltpu.sync_copy(acc, zv)
                pltpu.sync_copy(nxt_hbm, qv)

                def addq(g, carry):
                    zv[pl.ds(g * L, L)] = (zv[pl.ds(g * L, L)]
                                           + qv[pl.ds(g * L, L)])
                    return carry
                lax.fori_loop(0, N // L, addq, 0)
                plsc.subcore_barrier()

                @pl.when(s == 0)
                def _():
                    pltpu.sync_copy(zinit, acc)

                plsc.subcore_barrier()
            else:
                @pl.when(s == 0)
                def _():
                    pltpu.sync_copy(acc, zv)
                    pltpu.sync_copy(nxt_hbm, qv)

                    def addq(g, carry):
                        zv[pl.ds(g * L, L)] = (zv[pl.ds(g * L, L)]
                                               + qv[pl.ds(g * L, L)])
                        return carry
                    lax.fori_loop(0, N // L, addq, 0)
                    pltpu.sync_copy(zv, out_hbm)


_sc_layer3 = pl.kernel(
    _sc_layer3_body,
    out_type=jax.ShapeDtypeStruct((N,), _f32),
    mesh=_mesh,
    compiler_params=pltpu.CompilerParams(needs_layout_passes=False),
    scratch_types=[
        pltpu.VMEM((N,), _f32),
        pltpu.VMEM((N,), _f32),
        pltpu.VMEM((EPT3,), _i32),
        pltpu.VMEM((EPT3,), _i32),
        pltpu.VMEM((EPT3,), _f32),
        pltpu.VMEM((EPT3,), _f32),
        pltpu.VMEM((N,), _f32),
        pltpu.VMEM_SHARED((N,), _f32),
    ],
)


def _tc_proj1_body(x_ref, w_ref, u0, u1, u2, u3):
    x = x_ref[...]
    for k, o in enumerate((u0, u1, u2, u3)):
        o[...] = jnp.dot(x, w_ref[k], preferred_element_type=_f32)


def _tc_proj1(x, W1):
    return pl.pallas_call(
        _tc_proj1_body,
        out_shape=tuple(jax.ShapeDtypeStruct((N, H), _f32) for _ in range(K + 1)),
    )(x, W1)


def _tc_act_proj_body(u0_ref, t_ref, b_ref, a_ref, w_ref, o0, o1, o2, o3):
    h = u0_ref[...] + t_ref[...] + b_ref[...]
    a = a_ref[0, 0]
    h = jnp.where(h >= 0.0, h, a * h)
    for k, o in enumerate((o0, o1, o2, o3)):
        o[...] = jnp.dot(h, w_ref[k], preferred_element_type=_f32)


def _tc_act_proj(u0, t, b, a, W):
    return pl.pallas_call(
        _tc_act_proj_body,
        out_shape=tuple(jax.ShapeDtypeStruct((N, H), _f32) for _ in range(K + 1)),
    )(u0, t, b, a, W)


def _tc_act_proj3_body(v0_ref, t_ref, b_ref, a_ref, w_ref, b3_ref,
                       o0, o1, o2, o3):
    h = v0_ref[...] + t_ref[...] + b_ref[...]
    a = a_ref[0, 0]
    h = jnp.where(h >= 0.0, h, a * h)
    for k, o in enumerate((o0, o1, o2, o3)):
        o[...] = jnp.dot(h, w_ref[k, :, 0], preferred_element_type=_f32)
    o0[...] = o0[...] + b3_ref[0, 0]


def _tc_act_proj3(v0, t, b, a, W, b3):
    return pl.pallas_call(
        _tc_act_proj3_body,
        out_shape=tuple(jax.ShapeDtypeStruct((N,), _f32) for _ in range(K + 1)),
    )(v0, t, b, a, W, b3)


def kernel(x, edge_index, edge_attr, W1, b1, W2, b2, W3, b3, a1, a2):
    row = edge_index[0]
    col = edge_index[1]
    a1r = a1.reshape(1, 1)
    a2r = a2.reshape(1, 1)
    b3r = b3.reshape(1, 1)

    nrm = _sc_gcnnorm(row, col, edge_attr)
    u0, u1, u2, u3 = _tc_proj1(x, W1)

    t1, _z1, _p1, _f1 = _sc_layer16(u3, u2, u1, row, col, nrm)
    v0, v1, v2, v3 = _tc_act_proj(u0, t1, b1, a1r, W2)

    t2, _z2, _p2, _f2 = _sc_layer16(v3, v2, v1, row, col, nrm)
    s0b, s1, s2, s3 = _tc_act_proj3(v0, t2, b2, a2r, W3, b3r)

    out = _sc_layer3(s3, s2, s1, s0b, row, col, nrm)
    return out.reshape(N, 1)
